# baseline (speedup 1.0000x reference)
import functools

import jax
import jax.numpy as jnp
from jax import lax
from jax.experimental import pallas as pl
from jax.experimental.pallas import tpu as pltpu
from jax.experimental.pallas import tpu_sc as plsc

_NC = 2
_NS = 16
_NW = _NC * _NS
_G = 128
_WIN = 400
_PDIM = 128


def _sc_gather(idx_flat, table):
    n = idx_flat.shape[0]
    ids_per_worker = n // _NW
    n_win = ids_per_worker // _WIN
    chunks = []
    off = 0
    while off < _WIN:
        c = min(_G, _WIN - off)
        chunks.append((off, c))
        off += c
    mesh = plsc.VectorSubcoreMesh(core_axis_name="c", subcore_axis_name="s")

    @functools.partial(
        pl.kernel,
        mesh=mesh,
        out_type=jax.ShapeDtypeStruct((n, _PDIM), jnp.float32),
        scratch_types=[
            pltpu.VMEM((2, _WIN), jnp.int32),
            pltpu.VMEM((2, _WIN, _PDIM), jnp.float32),
            pltpu.SemaphoreType.DMA((2,)),
            pltpu.SemaphoreType.DMA((2,)),
            pltpu.SemaphoreType.DMA((2,)),
        ],
        compiler_params=pltpu.CompilerParams(use_tc_tiling_on_sc=False),
    )
    def gather_kernel(idx_hbm, table_hbm, out_hbm, idx_v, rows_v, sem_i,
                      sem_g, sem_o):
        wid = lax.axis_index("s") * _NC + lax.axis_index("c")
        id_base = wid * ids_per_worker

        def issue_idx(w, b):
            pltpu.async_copy(
                idx_hbm.at[pl.ds(id_base + w * _WIN, _WIN)],
                idx_v.at[b], sem_i.at[b])

        def wait_idx(b):
            pltpu.make_async_copy(
                idx_hbm.at[pl.ds(0, _WIN)], idx_v.at[b], sem_i.at[b]).wait()

        def issue_gathers(b):
            for (o, c) in chunks:
                pltpu.async_copy(
                    table_hbm.at[idx_v.at[b, pl.ds(o, c)]],
                    rows_v.at[b, pl.ds(o, c)], sem_g.at[b])

        def wait_gathers(b):
            for (o, c) in chunks:
                pltpu.make_async_copy(
                    table_hbm.at[pl.ds(0, c)],
                    rows_v.at[b, pl.ds(o, c)], sem_g.at[b]).wait()

        def issue_out(w, b):
            pltpu.async_copy(
                rows_v.at[b],
                out_hbm.at[pl.ds(id_base + w * _WIN, _WIN)],
                sem_o.at[b])

        def wait_out(b):
            pltpu.make_async_copy(
                rows_v.at[b], out_hbm.at[pl.ds(0, _WIN)],
                sem_o.at[b]).wait()

        issue_idx(0, 0)
        issue_idx(1, 1)
        wait_idx(0)
        issue_gathers(0)

        @pl.loop(0, n_win - 2, step=2)
        def _(g):
            wait_idx(1)

            @pl.when(g > 0)
            def _():
                wait_out(1)

            issue_gathers(1)
            wait_gathers(0)
            issue_out(g, 0)
            issue_idx(g + 2, 0)

            wait_idx(0)
            wait_out(0)
            issue_gathers(0)
            wait_gathers(1)
            issue_out(g + 1, 1)
            issue_idx(g + 3, 1)

        wait_idx(1)
        wait_out(1)
        issue_gathers(1)
        wait_gathers(0)
        issue_out(n_win - 2, 0)
        wait_gathers(1)
        issue_out(n_win - 1, 1)
        wait_out(0)
        wait_out(1)

    return gather_kernel(idx_flat, table)


def kernel(x, weight):
    b, s = x.shape
    dim = weight.shape[1]
    idx_flat = x.reshape(b * s).astype(jnp.int32)
    table = jnp.pad(weight, ((0, 0), (0, _PDIM - dim)))
    flat = _sc_gather(idx_flat, table)
    return flat.reshape(b, s, _PDIM)[:, :, :dim]


# speedup vs baseline: 1.4031x; 1.4031x over previous
import functools

import jax
import jax.numpy as jnp
from jax import lax
from jax.experimental import pallas as pl
from jax.experimental.pallas import tpu as pltpu
from jax.experimental.pallas import tpu_sc as plsc

_NC = 2
_NS = 16
_NW = _NC * _NS
_G = 128
_WIN = 400
_PDIM = 128


def _sc_gather(idx_flat, table, seq, pseq):
    n = idx_flat.shape[0]
    batch = n // seq
    bpw = _WIN // seq
    ids_per_worker = n // _NW
    batches_per_worker = batch // _NW
    n_win = ids_per_worker // _WIN
    n_out = batch * pseq
    chunks = []
    off = 0
    while off < _WIN:
        c = min(_G, _WIN - off)
        chunks.append((off, c))
        off += c
    mesh = plsc.VectorSubcoreMesh(core_axis_name="c", subcore_axis_name="s")

    @functools.partial(
        pl.kernel,
        mesh=mesh,
        out_type=jax.ShapeDtypeStruct((n_out, _PDIM), jnp.float32),
        scratch_types=[
            pltpu.VMEM((2, _WIN), jnp.int32),
            pltpu.VMEM((2, _WIN, _PDIM), jnp.float32),
            pltpu.SemaphoreType.DMA((2,)),
            pltpu.SemaphoreType.DMA((2,)),
            pltpu.SemaphoreType.DMA((2,)),
        ],
        compiler_params=pltpu.CompilerParams(use_tc_tiling_on_sc=False),
    )
    def gather_kernel(idx_hbm, table_hbm, out_hbm, idx_v, rows_v, sem_i,
                      sem_g, sem_o):
        wid = lax.axis_index("s") * _NC + lax.axis_index("c")
        id_base = wid * ids_per_worker
        batch_base = wid * batches_per_worker

        def issue_idx(w, b):
            pltpu.async_copy(
                idx_hbm.at[pl.ds(id_base + w * _WIN, _WIN)],
                idx_v.at[b], sem_i.at[b])

        def wait_idx(b):
            pltpu.make_async_copy(
                idx_hbm.at[pl.ds(0, _WIN)], idx_v.at[b], sem_i.at[b]).wait()

        def issue_gathers(b):
            for (o, c) in chunks:
                pltpu.async_copy(
                    table_hbm.at[idx_v.at[b, pl.ds(o, c)]],
                    rows_v.at[b, pl.ds(o, c)], sem_g.at[b])

        def wait_gathers(b):
            for (o, c) in chunks:
                pltpu.make_async_copy(
                    table_hbm.at[pl.ds(0, c)],
                    rows_v.at[b, pl.ds(o, c)], sem_g.at[b]).wait()

        def issue_out(w, b):
            for j in range(bpw):
                pltpu.async_copy(
                    rows_v.at[b, pl.ds(j * seq, seq)],
                    out_hbm.at[pl.ds((batch_base + w * bpw + j) * pseq, seq)],
                    sem_o.at[b])

        def wait_out(b):
            for j in range(bpw):
                pltpu.make_async_copy(
                    rows_v.at[b, pl.ds(j * seq, seq)],
                    out_hbm.at[pl.ds(0, seq)], sem_o.at[b]).wait()

        issue_idx(0, 0)
        issue_idx(1, 1)
        wait_idx(0)
        issue_gathers(0)

        @pl.loop(0, n_win - 2, step=2)
        def _(g):
            wait_idx(1)

            @pl.when(g > 0)
            def _():
                wait_out(1)

            issue_gathers(1)
            wait_gathers(0)
            issue_out(g, 0)
            issue_idx(g + 2, 0)

            wait_idx(0)
            wait_out(0)
            issue_gathers(0)
            wait_gathers(1)
            issue_out(g + 1, 1)
            issue_idx(g + 3, 1)

        wait_idx(1)
        wait_out(1)
        issue_gathers(1)
        wait_gathers(0)
        issue_out(n_win - 2, 0)
        wait_gathers(1)
        issue_out(n_win - 1, 1)
        wait_out(0)
        wait_out(1)

    return gather_kernel(idx_flat, table)


_PSEQ = 56


def kernel(x, weight):
    b, s = x.shape
    dim = weight.shape[1]
    idx_flat = x.reshape(b * s).astype(jnp.int32)
    table = jnp.pad(weight, ((0, 0), (0, _PDIM - dim)))
    flat = _sc_gather(idx_flat, table, s, _PSEQ)
    return flat.reshape(b, _PSEQ, _PDIM)[:, :s, :dim]


# speedup vs baseline: 1.4032x; 1.0001x over previous
import functools

import jax
import jax.numpy as jnp
from jax import lax
from jax.experimental import pallas as pl
from jax.experimental.pallas import tpu as pltpu
from jax.experimental.pallas import tpu_sc as plsc

_NC = 2
_NS = 16
_NW = _NC * _NS
_G = 128
_WIN = 400
_PDIM = 128


def _sc_gather(idx_flat, table, seq, pseq):
    n = idx_flat.shape[0]
    batch = n // seq
    bpw = _WIN // seq
    ids_per_worker = n // _NW
    batches_per_worker = batch // _NW
    n_win = ids_per_worker // _WIN
    n_out = batch * pseq
    chunks = []
    off = 0
    while off < _WIN:
        c = min(_G, _WIN - off)
        chunks.append((off, c))
        off += c
    mesh = plsc.VectorSubcoreMesh(core_axis_name="c", subcore_axis_name="s")

    @functools.partial(
        pl.kernel,
        mesh=mesh,
        out_type=jax.ShapeDtypeStruct((n_out, _PDIM), jnp.float32),
        scratch_types=[
            pltpu.VMEM((2, _WIN), jnp.int32),
            pltpu.VMEM((2, _WIN, _PDIM), jnp.float32),
            pltpu.SemaphoreType.DMA((2,)),
            pltpu.SemaphoreType.DMA((2,)),
            pltpu.SemaphoreType.DMA((2,)),
        ],
        compiler_params=pltpu.CompilerParams(use_tc_tiling_on_sc=False),
    )
    def gather_kernel(idx_hbm, table_hbm, out_hbm, idx_v, rows_v, sem_i,
                      sem_g, sem_o):
        wid = lax.axis_index("s") * _NC + lax.axis_index("c")
        id_base = wid * ids_per_worker
        batch_base = wid * batches_per_worker

        def issue_idx(w, b):
            pltpu.async_copy(
                idx_hbm.at[pl.ds(id_base + w * _WIN, _WIN)],
                idx_v.at[b], sem_i.at[b])

        def wait_idx(b):
            pltpu.make_async_copy(
                idx_hbm.at[pl.ds(0, _WIN)], idx_v.at[b], sem_i.at[b]).wait()

        def issue_gathers(b):
            for (o, c) in chunks:
                pltpu.async_copy(
                    table_hbm.at[idx_v.at[b, pl.ds(o, c)]],
                    rows_v.at[b, pl.ds(o, c)], sem_g.at[b])

        def wait_gathers(b):
            for (o, c) in chunks:
                pltpu.make_async_copy(
                    table_hbm.at[pl.ds(0, c)],
                    rows_v.at[b, pl.ds(o, c)], sem_g.at[b]).wait()

        def issue_out(w, b):
            for j in range(bpw):
                pltpu.async_copy(
                    rows_v.at[b, pl.ds(j * seq, seq)],
                    out_hbm.at[pl.ds((batch_base + w * bpw + j) * pseq, seq)],
                    sem_o.at[b])

        def wait_out(b):
            for j in range(bpw):
                pltpu.make_async_copy(
                    rows_v.at[b, pl.ds(j * seq, seq)],
                    out_hbm.at[pl.ds(0, seq)], sem_o.at[b]).wait()

        issue_idx(0, 0)
        issue_idx(1, 1)
        wait_idx(0)
        issue_gathers(0)

        @pl.loop(0, n_win - 2, step=2)
        def _(g):
            wait_idx(1)

            @pl.when(g > 0)
            def _():
                wait_out(1)

            issue_gathers(1)
            wait_gathers(0)
            issue_out(g, 0)
            issue_idx(g + 2, 0)

            wait_idx(0)
            wait_out(0)
            issue_gathers(0)
            wait_gathers(1)
            issue_out(g + 1, 1)
            issue_idx(g + 3, 1)

        wait_idx(1)
        wait_out(1)
        issue_gathers(1)
        wait_gathers(0)
        issue_out(n_win - 2, 0)
        wait_gathers(1)
        issue_out(n_win - 1, 1)
        wait_out(0)
        wait_out(1)

    return gather_kernel(idx_flat, table)


_PSEQ = 56


def kernel(x, weight):
    b, s = x.shape
    dim = weight.shape[1]
    idx_flat = x.reshape(b * s).astype(jnp.int32)
    table = jnp.concatenate(
        [weight, jnp.zeros((weight.shape[0], _PDIM - dim), jnp.float32)],
        axis=1)
    flat = _sc_gather(idx_flat, table, s, _PSEQ)
    return flat.reshape(b, _PSEQ, _PDIM)[:, :s, :dim]
